# baseline (speedup 1.0000x reference)
import functools

import jax
import jax.numpy as jnp
from jax import lax
from jax.experimental import pallas as pl
from jax.experimental.pallas import tpu as pltpu
from jax.experimental.pallas import tpu_sc as plsc

_DIM = 64
_SEQ = 200
_NC = 2
_NS = 16
_NW = _NC * _NS
_L = 16
_CHUNK = 104


def _emb_mean_body(bpw, text_hbm, table_hbm, out_hbm, idx_v, rows_v, out_v,
                   sem):
    wid = lax.axis_index("s") * _NC + lax.axis_index("c")
    b0 = wid * bpw

    pltpu.sync_copy(text_hbm.at[pl.ds(b0 * _SEQ, bpw * _SEQ)], idx_v)

    def body(i, _):
        cp0 = pltpu.async_copy(
            table_hbm.at[idx_v.at[pl.ds(i * _SEQ, _CHUNK)]],
            rows_v.at[pl.ds(0, _CHUNK)], sem)
        cp1 = pltpu.async_copy(
            table_hbm.at[idx_v.at[pl.ds(i * _SEQ + _CHUNK, _SEQ - _CHUNK)]],
            rows_v.at[pl.ds(_CHUNK, _SEQ - _CHUNK)], sem)
        cp0.wait()
        cp1.wait()

        def rbody(r, acc):
            return tuple(acc[k] + rows_v[r, pl.ds(k * _L, _L)]
                         for k in range(_DIM // _L))

        acc = lax.fori_loop(
            0, _SEQ, rbody,
            tuple(jnp.zeros((_L,), jnp.float32) for _ in range(_DIM // _L)))
        for k in range(_DIM // _L):
            out_v[i, pl.ds(k * _L, _L)] = acc[k] * (1.0 / _SEQ)
        return 0

    lax.fori_loop(0, bpw, body, 0)
    pltpu.sync_copy(out_v, out_hbm.at[pl.ds(b0, bpw)])


@functools.partial(jax.jit, static_argnames=("batch",))
def _emb_mean(idx_flat, table, batch):
    bpw = batch // _NW
    mesh = plsc.VectorSubcoreMesh(
        core_axis_name="c", subcore_axis_name="s",
        num_cores=_NC, num_subcores=_NS)
    return pl.kernel(
        functools.partial(_emb_mean_body, bpw),
        out_type=jax.ShapeDtypeStruct((batch, _DIM), jnp.float32),
        mesh=mesh,
        compiler_params=pltpu.CompilerParams(use_tc_tiling_on_sc=False),
        scratch_types=[
            pltpu.VMEM((bpw * _SEQ,), jnp.int32),
            pltpu.VMEM((_SEQ, _DIM), jnp.float32),
            pltpu.VMEM((bpw, _DIM), jnp.float32),
            pltpu.SemaphoreType.DMA,
        ],
    )(idx_flat, table)


def kernel(text, table):
    batch = text.shape[0]
    idx_flat = text.astype(jnp.int32).reshape(-1)
    return _emb_mean(idx_flat, table, batch)


# speedup vs baseline: 1.1741x; 1.1741x over previous
import functools

import jax
import jax.numpy as jnp
from jax import lax
from jax.experimental import pallas as pl
from jax.experimental.pallas import tpu as pltpu
from jax.experimental.pallas import tpu_sc as plsc

_DIM = 64
_SEQ = 200
_NC = 2
_NS = 16
_NW = _NC * _NS
_L = 16
_CHUNK = 104


_UNROLL = 8


def _emb_mean_body(bpw, text_hbm, table_hbm, out_hbm, idx_v, rows_v, out_v,
                   sem_a, sem_b):
    wid = lax.axis_index("s") * _NC + lax.axis_index("c")
    b0 = wid * bpw

    pltpu.sync_copy(text_hbm.at[pl.ds(b0 * _SEQ, bpw * _SEQ)], idx_v)

    buf_a = rows_v.at[0]
    buf_b = rows_v.at[1]

    def fire(i, buf, sem):
        pltpu.async_copy(
            table_hbm.at[idx_v.at[pl.ds(i * _SEQ, _CHUNK)]],
            buf.at[pl.ds(0, _CHUNK)], sem)
        pltpu.async_copy(
            table_hbm.at[idx_v.at[pl.ds(i * _SEQ + _CHUNK, _SEQ - _CHUNK)]],
            buf.at[pl.ds(_CHUNK, _SEQ - _CHUNK)], sem)

    def drain(buf, sem):
        pltpu.make_async_copy(table_hbm.at[pl.ds(0, _SEQ)], buf, sem).wait()

    def reduce_into(buf, i):
        def rbody(g, acc):
            accs = list(acc)
            for u in range(_UNROLL):
                r = g * _UNROLL + u
                for k in range(_DIM // _L):
                    accs[k] = accs[k] + buf[r, pl.ds(k * _L, _L)]
            return tuple(accs)

        acc = lax.fori_loop(
            0, _SEQ // _UNROLL, rbody,
            tuple(jnp.zeros((_L,), jnp.float32) for _ in range(_DIM // _L)))
        for k in range(_DIM // _L):
            out_v[i, pl.ds(k * _L, _L)] = acc[k] * (1.0 / _SEQ)

    fire(0, buf_a, sem_a)

    def body(j, _):
        i = j * 2
        fire(i + 1, buf_b, sem_b)
        drain(buf_a, sem_a)
        reduce_into(buf_a, i)

        @pl.when(i + 2 < bpw)
        def _():
            fire(i + 2, buf_a, sem_a)

        drain(buf_b, sem_b)
        reduce_into(buf_b, i + 1)
        return 0

    lax.fori_loop(0, bpw // 2, body, 0)
    pltpu.sync_copy(out_v, out_hbm.at[pl.ds(b0, bpw)])


@functools.partial(jax.jit, static_argnames=("batch",))
def _emb_mean(idx_flat, table, batch):
    bpw = batch // _NW
    mesh = plsc.VectorSubcoreMesh(
        core_axis_name="c", subcore_axis_name="s",
        num_cores=_NC, num_subcores=_NS)
    return pl.kernel(
        functools.partial(_emb_mean_body, bpw),
        out_type=jax.ShapeDtypeStruct((batch, _DIM), jnp.float32),
        mesh=mesh,
        compiler_params=pltpu.CompilerParams(use_tc_tiling_on_sc=False),
        scratch_types=[
            pltpu.VMEM((bpw * _SEQ,), jnp.int32),
            pltpu.VMEM((2, _SEQ, _DIM), jnp.float32),
            pltpu.VMEM((bpw, _DIM), jnp.float32),
            pltpu.SemaphoreType.DMA,
            pltpu.SemaphoreType.DMA,
        ],
    )(idx_flat, table)


def kernel(text, table):
    batch = text.shape[0]
    idx_flat = text.astype(jnp.int32).reshape(-1)
    return _emb_mean(idx_flat, table, batch)
